# baseline (speedup 1.0000x reference)
import functools
import math

import jax
import jax.numpy as jnp
from jax import lax
from jax.experimental import pallas as pl
from jax.experimental.pallas import tpu as pltpu
from jax.experimental.pallas import tpu_sc as plsc

EMB = 64
SCALE = math.sqrt(EMB)

NUM_WORKERS = 32
OCHUNK = 128
TCHUNK = 2 * OCHUNK
GROUPS = TCHUNK // 16


def _make_sc_lookup(B, b_per_w, r_per_w, n_chunks):
    mesh = plsc.VectorSubcoreMesh(core_axis_name="c", subcore_axis_name="s")

    @functools.partial(
        pl.kernel,
        mesh=mesh,
        out_type=jax.ShapeDtypeStruct((B // 2, 2 * EMB), jnp.float32),
        scratch_types=[
            pltpu.VMEM((b_per_w,), jnp.int32),
            pltpu.VMEM((TCHUNK,), jnp.int32),
            pltpu.VMEM((TCHUNK,), jnp.int32),
            pltpu.VMEM((TCHUNK, 128), jnp.float32),
            pltpu.VMEM((TCHUNK, 128), jnp.float32),
            pltpu.VMEM((OCHUNK, 128), jnp.float32),
            pltpu.VMEM((OCHUNK, 128), jnp.float32),
            pltpu.SemaphoreType.DMA,
            pltpu.SemaphoreType.DMA,
            pltpu.SemaphoreType.DMA,
            pltpu.SemaphoreType.DMA,
        ],
        compiler_params=pltpu.CompilerParams(needs_layout_passes=False),
    )
    def lookup(tbl2_hbm, tok_hbm, out_hbm,
               tok_v, ridx0, ridx1, gbuf0, gbuf1, obuf0, obuf1,
               gsem0, gsem1, ssem0, ssem1):
        ridx = (ridx0, ridx1)
        gbuf = (gbuf0, gbuf1)
        obuf = (obuf0, obuf1)
        gsem = (gsem0, gsem1)
        ssem = (ssem0, ssem1)
        wid = lax.axis_index("s") * 2 + lax.axis_index("c")
        tbase = wid * b_per_w
        obase = wid * r_per_w
        pltpu.sync_copy(tok_hbm.at[pl.ds(tbase, b_per_w)], tok_v)

        iota16 = lax.iota(jnp.int32, 16)

        def compute_ridx(g, b):
            def q_body(q, carry):
                tokv = tok_v[pl.ds(g * TCHUNK + q * 16, 16)]
                ridx[b][pl.ds(q * 16, 16)] = lax.shift_right_logical(tokv, 1)
                return carry

            lax.fori_loop(0, GROUPS, q_body, 0, unroll=4)

        def gather(g, b):
            pltpu.async_copy(tbl2_hbm.at[ridx[b]], gbuf[b], gsem[b])

        def wait_gather(g, b):
            pltpu.make_async_copy(tbl2_hbm.at[ridx[b]], gbuf[b],
                                  gsem[b]).wait()

        def scatter(g, b):
            pltpu.async_copy(
                obuf[b], out_hbm.at[pl.ds(obase + g * OCHUNK, OCHUNK)],
                ssem[b])

        def wait_scatter(g, b):
            pltpu.make_async_copy(
                obuf[b], out_hbm.at[pl.ds(obase + g * OCHUNK, OCHUNK)],
                ssem[b]).wait()

        def select_scale(g, b):
            gb = gbuf[b]
            ob = obuf[b]

            def group_body(grp, carry):
                jv = grp * 16 + iota16
                tokv = tok_v[pl.ds(g * TCHUNK + grp * 16, 16)]
                hv = lax.shift_left(jnp.bitwise_and(tokv, 1), 6)
                rowv_out = lax.shift_right_logical(jv, 1)
                colv_base = lax.shift_left(jnp.bitwise_and(jv, 1), 6)

                @plsc.parallel_loop(0, EMB, unroll=4)
                def _(c):
                    x = plsc.load_gather(gb, [jv, hv + c])
                    plsc.store_scatter(ob, [rowv_out, colv_base + c],
                                       x * SCALE)

                return carry

            lax.fori_loop(0, GROUPS, group_body, 0)

        def step(g, b, first, last):
            other = 1 - b
            if not first:
                wait_scatter(g - 1, other)
            if not last:
                compute_ridx(g + 1, other)
                gather(g + 1, other)
            wait_gather(g, b)
            select_scale(g, b)
            scatter(g, b)

        compute_ridx(0, 0)
        gather(0, 0)
        step(0, 0, first=True, last=False)
        step(1, 1, first=False, last=False)

        def outer(t, carry):
            g = 2 * t
            step(g, 0, first=False, last=False)
            step(g + 1, 1, first=False, last=False)
            return carry

        lax.fori_loop(1, n_chunks // 2 - 1, outer, 0)
        step(n_chunks - 2, 0, first=False, last=False)
        step(n_chunks - 1, 1, first=False, last=True)
        wait_scatter(n_chunks - 1, 1)

    return lookup


def kernel(token_sequences, table):
    Bseq, L = token_sequences.shape
    V, D = table.shape
    B = Bseq * L
    b_per_w = B // NUM_WORKERS
    r_per_w = b_per_w // 2
    n_chunks = b_per_w // TCHUNK
    idx_flat = token_sequences.reshape(B)
    tbl2 = table.reshape(V // 2, 2 * D)
    out2 = _make_sc_lookup(B, b_per_w, r_per_w, n_chunks)(tbl2, idx_flat)
    return out2.reshape(Bseq, L, D)


# speedup vs baseline: 1.7874x; 1.7874x over previous
import functools
import math

import jax
import jax.numpy as jnp
from jax import lax
from jax.experimental import pallas as pl
from jax.experimental.pallas import tpu as pltpu
from jax.experimental.pallas import tpu_sc as plsc

EMB = 64
SCALE = math.sqrt(EMB)

NUM_WORKERS = 32
SEQ_CHUNK = 4
L = 200
CHUNK = SEQ_CHUNK * L


def _make_sc_lookup(Bseq, b_per_w, n_chunks):
    mesh = plsc.VectorSubcoreMesh(core_axis_name="c", subcore_axis_name="s")

    @functools.partial(
        pl.kernel,
        mesh=mesh,
        out_type=jax.ShapeDtypeStruct((Bseq, L, EMB), jnp.float32),
        scratch_types=[
            pltpu.VMEM((CHUNK,), jnp.int32),
            pltpu.VMEM((CHUNK,), jnp.int32),
            pltpu.VMEM((CHUNK, EMB), jnp.float32),
            pltpu.VMEM((CHUNK, EMB), jnp.float32),
            pltpu.SemaphoreType.DMA,
            pltpu.SemaphoreType.DMA,
            pltpu.SemaphoreType.DMA,
            pltpu.SemaphoreType.DMA,
        ],
        compiler_params=pltpu.CompilerParams(use_tc_tiling_on_sc=False),
    )
    def lookup(table_hbm, idx_hbm, out_hbm,
               iring0, iring1, rows0, rows1,
               gsem0, gsem1, ssem0, ssem1):
        iring = (iring0, iring1)
        rows = (rows0, rows1)
        gsem = (gsem0, gsem1)
        ssem = (ssem0, ssem1)
        wid = lax.axis_index("s") * 2 + lax.axis_index("c")
        tbase = wid * b_per_w
        sbase = wid * (b_per_w // L)

        def load_idx(g, b):
            pltpu.sync_copy(idx_hbm.at[pl.ds(tbase + g * CHUNK, CHUNK)],
                            iring[b])

        def gather(g, b):
            pltpu.async_copy(table_hbm.at[iring[b]], rows[b], gsem[b])

        def wait_gather(g, b):
            pltpu.make_async_copy(table_hbm.at[iring[b]], rows[b],
                                  gsem[b]).wait()

        def scatter(g, b):
            for i in range(SEQ_CHUNK):
                pltpu.async_copy(
                    rows[b].at[pl.ds(i * L, L)],
                    out_hbm.at[sbase + g * SEQ_CHUNK + i], ssem[b])

        def wait_scatter(g, b):
            for i in range(SEQ_CHUNK):
                pltpu.make_async_copy(
                    rows[b].at[pl.ds(i * L, L)],
                    out_hbm.at[sbase + g * SEQ_CHUNK + i], ssem[b]).wait()

        def scale(b):
            buf = rows[b]

            @plsc.parallel_loop(0, CHUNK, unroll=4)
            def _(i):
                for j in range(EMB // 16):
                    sl = pl.ds(j * 16, 16)
                    buf[i, sl] = buf[i, sl] * SCALE

        def step(g, b, first, last):
            other = 1 - b
            if not first:
                wait_scatter(g - 1, other)
            if not last:
                load_idx(g + 1, other)
                gather(g + 1, other)
            wait_gather(g, b)
            scale(b)
            scatter(g, b)

        load_idx(0, 0)
        gather(0, 0)
        step(0, 0, first=True, last=False)
        step(1, 1, first=False, last=False)

        def outer(t, carry):
            g = 2 * t
            step(g, 0, first=False, last=False)
            step(g + 1, 1, first=False, last=False)
            return carry

        lax.fori_loop(1, n_chunks // 2 - 1, outer, 0)
        step(n_chunks - 2, 0, first=False, last=False)
        step(n_chunks - 1, 1, first=False, last=True)
        wait_scatter(n_chunks - 1, 1)

    return lookup


def kernel(token_sequences, table):
    Bseq, Lx = token_sequences.shape
    B = Bseq * Lx
    b_per_w = B // NUM_WORKERS
    n_chunks = b_per_w // CHUNK
    idx_flat = token_sequences.reshape(B)
    return _make_sc_lookup(Bseq, b_per_w, n_chunks)(table, idx_flat)
